# baseline (speedup 1.0000x reference)
import functools

import jax
import jax.numpy as jnp
from jax import lax
from jax.experimental import pallas as pl
from jax.experimental.pallas import tpu as pltpu
from jax.experimental.pallas import tpu_sc as plsc

N = 10000
H = 128
EIN = 16
E = 320000
PADE = 7680
EG = E + PADE
E2 = 2 * EG
N2 = 2 * N
NPAD = 10240
NG = 256
NW = 32
CH = 128


NB = 5
LAG = NB - 2


def _sc_gather(h2, idx):
    mesh = plsc.VectorSubcoreMesh(core_axis_name="c", subcore_axis_name="s")
    epw = E2 // NW
    nch = epw // CH

    dt = h2.dtype
    kw = h2.shape[1]

    def body(h_hbm, idx_hbm, out_hbm, idx_all, rows, gsem, osem):
        wid = lax.axis_index("s") * 2 + lax.axis_index("c")
        base = wid * epw
        pltpu.sync_copy(idx_hbm.at[pl.ds(base, epw)], idx_all)

        def gstart(p):
            b = lax.rem(p, NB)
            pltpu.async_copy(h_hbm.at[idx_all.at[pl.ds(p * CH, CH)]],
                             rows.at[b], gsem.at[b])

        def gwait(p):
            b = lax.rem(p, NB)
            pltpu.make_async_copy(h_hbm.at[pl.ds(0, CH)], rows.at[b],
                                  gsem.at[b]).wait()

        def ostart(p):
            b = lax.rem(p, NB)
            pltpu.async_copy(rows.at[b],
                             out_hbm.at[pl.ds(base + p * CH, CH)],
                             osem.at[b])

        def owait(p):
            b = lax.rem(p, NB)
            pltpu.make_async_copy(rows.at[b],
                                  out_hbm.at[pl.ds(base + p * CH, CH)],
                                  osem.at[b]).wait()

        def step(p, carry):
            @pl.when(p < nch)
            def _():
                @pl.when(p >= NB)
                def _():
                    owait(p - NB)
                gstart(p)

            @pl.when(p >= LAG)
            def _():
                gwait(p - LAG)
                ostart(p - LAG)
            return carry

        lax.fori_loop(0, nch + LAG, step, 0)
        for q in range(nch - NB, nch):
            owait(q)

    f = pl.kernel(
        body,
        out_type=jax.ShapeDtypeStruct((E2, kw), dt),
        mesh=mesh,
        scratch_types=[
            pltpu.VMEM((epw,), jnp.int32),
            pltpu.VMEM((NB, CH, kw), dt),
            pltpu.SemaphoreType.DMA((NB,)),
            pltpu.SemaphoreType.DMA((NB,)),
        ],
    )
    return f(h2, idx)


def _sc_scatter(m, dst):
    mesh = plsc.VectorSubcoreMesh(core_axis_name="c", subcore_axis_name="s")
    ept = EG // 16
    nch = ept // CH
    rpt = NPAD // 16

    nb = 2

    def body(m_hbm, dst_hbm, out_hbm, idxb, mbuf, zbuf, acc,
             isem, lsem, ssem):
        c = lax.axis_index("c")
        s = lax.axis_index("s")

        def zb(i, carry):
            zbuf[i // 8, pl.ds((i % 8) * 16, 16)] = jnp.zeros((16,), jnp.float32)
            return carry

        lax.fori_loop(0, 32 * 8, zb, 0)

        def za(i, carry):
            pltpu.sync_copy(zbuf, acc.at[pl.ds(s * rpt + i * 32, 32)])
            return carry

        lax.fori_loop(0, rpt // 32, za, 0)
        plsc.subcore_barrier()

        base = c * EG + s * ept

        def istart(p):
            b = lax.rem(p, nb)
            pltpu.async_copy(dst_hbm.at[c, s, p], idxb.at[b], isem.at[b])

        def iwait(p):
            b = lax.rem(p, nb)
            pltpu.make_async_copy(dst_hbm.at[c, s, p], idxb.at[b],
                                  isem.at[b]).wait()

        def lstart(p):
            b = lax.rem(p, nb)
            pltpu.async_copy(m_hbm.at[pl.ds(base + p * CH, CH)],
                             mbuf.at[b], lsem.at[b])

        def lwait(p):
            b = lax.rem(p, nb)
            pltpu.make_async_copy(m_hbm.at[pl.ds(base + p * CH, CH)],
                                  mbuf.at[b], lsem.at[b]).wait()

        def sstart(p):
            b = lax.rem(p, nb)
            pltpu.async_copy(mbuf.at[b], acc.at[idxb.at[b]],
                             ssem.at[b], add=True)

        def swait(p):
            b = lax.rem(p, nb)
            pltpu.make_async_copy(mbuf.at[b], acc.at[pl.ds(0, CH)],
                                  ssem.at[b]).wait()

        def step(p, carry):
            @pl.when(p < nch)
            def _():
                @pl.when(p >= nb)
                def _():
                    swait(p - nb)
                istart(p)
                lstart(p)

            @pl.when(p >= 1)
            def _():
                iwait(p - 1)
                lwait(p - 1)
                sstart(p - 1)
            return carry

        lax.fori_loop(0, nch + 1, step, 0)
        swait(nch - 2)
        swait(nch - 1)
        plsc.subcore_barrier()

        pltpu.sync_copy(acc.at[pl.ds(s * rpt, rpt)],
                        out_hbm.at[c, pl.ds(s * rpt, rpt)])

    f = pl.kernel(
        body,
        out_type=jax.ShapeDtypeStruct((2, NPAD, H), jnp.float32),
        mesh=mesh,
        scratch_types=[
            pltpu.VMEM((nb, CH), jnp.int32),
            pltpu.VMEM((nb, CH, H), jnp.float32),
            pltpu.VMEM((32, H), jnp.float32),
            pltpu.VMEM_SHARED((NPAD, H), jnp.float32),
            pltpu.SemaphoreType.DMA((nb,)),
            pltpu.SemaphoreType.DMA((nb,)),
            pltpu.SemaphoreType.DMA((nb,)),
        ],
    )
    return f(m, dst)


def _pack_bf16_pair(o):
    obf = o.astype(jnp.bfloat16)
    lo = lax.bitcast_convert_type(obf[:, :H // 2], jnp.uint16).astype(jnp.uint32)
    hi = lax.bitcast_convert_type(obf[:, H // 2:], jnp.uint16).astype(jnp.uint32)
    return lax.bitcast_convert_type((hi << 16) | lo, jnp.int32)


def _unpack_bf16_pair(w):
    xw = lax.bitcast_convert_type(w, jnp.uint32)
    lo = lax.bitcast_convert_type(xw << 16, jnp.float32)
    hi = lax.bitcast_convert_type(xw & jnp.uint32(0xFFFF0000), jnp.float32)
    return jnp.concatenate([lo, hi], axis=1)


def _tc_linear_relu(x, wt, b):
    bm = 2000
    grid = x.shape[0] // bm

    def body(x_ref, w_ref, b_ref, o_ref, ob_ref):
        o = jnp.maximum(
            jnp.dot(x_ref[...], w_ref[...], preferred_element_type=jnp.float32)
            + b_ref[...], 0.0)
        o_ref[...] = o
        ob_ref[...] = _pack_bf16_pair(o)

    return pl.pallas_call(
        body,
        grid=(grid,),
        in_specs=[
            pl.BlockSpec((bm, x.shape[1]), lambda i: (i, 0)),
            pl.BlockSpec(wt.shape, lambda i: (0, 0)),
            pl.BlockSpec((1, wt.shape[1]), lambda i: (0, 0)),
        ],
        out_specs=[
            pl.BlockSpec((bm, wt.shape[1]), lambda i: (i, 0)),
            pl.BlockSpec((bm, wt.shape[1] // 2), lambda i: (i, 0)),
        ],
        out_shape=[
            jax.ShapeDtypeStruct((x.shape[0], wt.shape[1]), jnp.float32),
            jax.ShapeDtypeStruct((x.shape[0], wt.shape[1] // 2), jnp.int32),
        ],
    )(x, wt, b.reshape(1, -1))


def _tc_msg_mlp(hs, par, ea, w1h, w1e, b1, w2, b2):
    bm = 4096
    grid = E2 // bm

    def body(hs_ref, par_ref, ea_ref, w1h_ref, w1e_ref, b1_ref, w2_ref,
             b2_ref, o_ref):
        hp = hs_ref[...]
        sel = jnp.where(par_ref[...] != 0, hp[:, H // 2:], hp[:, :H // 2])
        m1 = jnp.dot(_unpack_bf16_pair(sel), w1h_ref[...],
                     preferred_element_type=jnp.float32)
        m1 = m1 + jnp.dot(ea_ref[...], w1e_ref[...],
                          preferred_element_type=jnp.float32)
        m1 = jnp.maximum(m1 + b1_ref[...], 0.0)
        o_ref[...] = (jnp.dot(m1, w2_ref[...], preferred_element_type=jnp.float32)
                      + b2_ref[...])

    return pl.pallas_call(
        body,
        grid=(grid,),
        in_specs=[
            pl.BlockSpec((bm, H), lambda i: (i, 0)),
            pl.BlockSpec((bm, 1), lambda i: (i, 0)),
            pl.BlockSpec((bm, EIN), lambda i: (i, 0)),
            pl.BlockSpec((H, H), lambda i: (0, 0)),
            pl.BlockSpec((EIN, H), lambda i: (0, 0)),
            pl.BlockSpec((1, H), lambda i: (0, 0)),
            pl.BlockSpec((H, H), lambda i: (0, 0)),
            pl.BlockSpec((1, H), lambda i: (0, 0)),
        ],
        out_specs=pl.BlockSpec((bm, H), lambda i: (i, 0)),
        out_shape=jax.ShapeDtypeStruct((E2, H), jnp.float32),
    )(hs, par, ea, w1h, w1e, b1.reshape(1, H), w2, b2.reshape(1, H))


def _tc_gru(agg, h, wih_t, whh_t, bih, bhh, bn_scale, bn_shift):
    bm = 2000
    grid = N2 // bm

    def body(agg_ref, h_ref, wih_ref, whh_ref, bih_ref, bhh_ref,
             bs_ref, bb_ref, o_ref, ob_ref):
        hv = h_ref[...]
        gi = (jnp.dot(agg_ref[...], wih_ref[...],
                      preferred_element_type=jnp.float32) + bih_ref[...])
        gh = (jnp.dot(hv, whh_ref[...],
                      preferred_element_type=jnp.float32) + bhh_ref[...])
        r = jax.nn.sigmoid(gi[:, :H] + gh[:, :H])
        z = jax.nn.sigmoid(gi[:, H:2 * H] + gh[:, H:2 * H])
        nn = jnp.tanh(gi[:, 2 * H:] + r * gh[:, 2 * H:])
        hn = (1.0 - z) * nn + z * hv
        o = hv + hn * bs_ref[...] + bb_ref[...]
        o_ref[...] = o
        ob_ref[...] = _pack_bf16_pair(o)

    return pl.pallas_call(
        body,
        grid=(grid,),
        in_specs=[
            pl.BlockSpec((bm, H), lambda i: (i, 0)),
            pl.BlockSpec((bm, H), lambda i: (i, 0)),
            pl.BlockSpec((H, 3 * H), lambda i: (0, 0)),
            pl.BlockSpec((H, 3 * H), lambda i: (0, 0)),
            pl.BlockSpec((1, 3 * H), lambda i: (0, 0)),
            pl.BlockSpec((1, 3 * H), lambda i: (0, 0)),
            pl.BlockSpec((1, H), lambda i: (0, 0)),
            pl.BlockSpec((1, H), lambda i: (0, 0)),
        ],
        out_specs=[
            pl.BlockSpec((bm, H), lambda i: (i, 0)),
            pl.BlockSpec((bm, H // 2), lambda i: (i, 0)),
        ],
        out_shape=[
            jax.ShapeDtypeStruct((N2, H), jnp.float32),
            jax.ShapeDtypeStruct((N2, H // 2), jnp.int32),
        ],
    )(agg, h, wih_t, whh_t, bih.reshape(1, -1), bhh.reshape(1, -1),
      bn_scale.reshape(1, H), bn_shift.reshape(1, H))


def _tc_readout(h, batch3d):
    bn = 1000
    grid = N2 // bn
    nseg = 2 * NG

    def body(h_ref, b_ref, bc_ref, zmean_ref, zmax_ref, accs, accm, accc):
        i = pl.program_id(0)

        @pl.when(i == 0)
        def _init():
            accs[...] = jnp.zeros_like(accs)
            accm[...] = jnp.full_like(accm, -1e30)
            accc[...] = jnp.zeros_like(accc)

        bv = b_ref[...].reshape(1, bn)
        hv = h_ref[...]
        seg = lax.broadcasted_iota(jnp.int32, (nseg, bn), 0)
        p = (seg == bv).astype(jnp.float32)
        accs[...] += jnp.dot(p, hv, preferred_element_type=jnp.float32)
        accc[...] += jnp.sum(p, axis=1, keepdims=True)

        g0 = jnp.min(bv)
        g1 = jnp.max(bv)

        bc = bc_ref[...]

        def gbody(g, carry):
            mask = bc == g
            hm = jnp.max(jnp.where(mask, hv, -1e30), axis=0, keepdims=True)
            accm[pl.ds(g, 1), :] = jnp.maximum(accm[pl.ds(g, 1), :], hm)
            return carry

        lax.fori_loop(g0, g1 + 1, gbody, 0)

        @pl.when(i == grid - 1)
        def _fin():
            cnt = accc[...]
            zmean_ref[...] = accs[...] / jnp.maximum(cnt, 1.0)
            zmax_ref[...] = jnp.where(cnt > 0.0, accm[...], 0.0)

    return pl.pallas_call(
        body,
        grid=(grid,),
        in_specs=[
            pl.BlockSpec((bn, H), lambda i: (i, 0)),
            pl.BlockSpec((1, 1, bn), lambda i: (i, 0, 0)),
            pl.BlockSpec((bn, 1), lambda i: (i, 0)),
        ],
        out_specs=[
            pl.BlockSpec((nseg, H), lambda i: (0, 0)),
            pl.BlockSpec((nseg, H), lambda i: (0, 0)),
        ],
        out_shape=[
            jax.ShapeDtypeStruct((nseg, H), jnp.float32),
            jax.ShapeDtypeStruct((nseg, H), jnp.float32),
        ],
        scratch_shapes=[
            pltpu.VMEM((nseg, H), jnp.float32),
            pltpu.VMEM((nseg, H), jnp.float32),
            pltpu.VMEM((nseg, 1), jnp.float32),
        ],
    )(h, batch3d, batch3d.reshape(N2, 1))


def _tc_head(z_mean, z_max, aux1, aux2, pk):
    def ln(x, g, b):
        mu = jnp.mean(x, axis=-1, keepdims=True)
        v = jnp.mean((x - mu) ** 2, axis=-1, keepdims=True)
        return (x - mu) * lax.rsqrt(v + 1e-5) * g + b

    def dot(a, b):
        return jnp.dot(a, b, preferred_element_type=jnp.float32)

    def body(zm_ref, zx_ref, a1_ref, a2_ref,
             wr_ref, br_ref, aw1_ref, ab1_ref, aw2_ref, ab2_ref,
             f1w_ref, f1b_ref, f2w_ref, f2b_ref, flg_ref, flb_ref,
             g121w_ref, g121b_ref, g122w_ref, g122b_ref,
             g211w_ref, g211b_ref, g212w_ref, g212b_ref,
             c1g_ref, c1b_ref, c2g_ref, c2b_ref,
             s1w_ref, s1b_ref, s2w_ref, s2b_ref,
             lw_ref, lb_ref, cd1w_ref, cd1b_ref, cd2w_ref, cd2b_ref,
             logits_ref, card_ref):
        zcat = jnp.concatenate([zm_ref[...], zx_ref[...]], axis=1)
        z = jnp.maximum(dot(zcat, wr_ref[...]) + br_ref[...], 0.0)
        z1g = z[:NG]
        z2g = z[NG:]
        za1 = jnp.maximum(dot(a1_ref[...], aw1_ref[...]) + ab1_ref[...], 0.0)
        za1 = jnp.maximum(dot(za1, aw2_ref[...]) + ab2_ref[...], 0.0)
        za2 = jnp.maximum(dot(a2_ref[...], aw1_ref[...]) + ab1_ref[...], 0.0)
        za2 = jnp.maximum(dot(za2, aw2_ref[...]) + ab2_ref[...], 0.0)

        def fuse(zg, za):
            gin = jnp.concatenate([zg, za], axis=1)
            gpre = jnp.maximum(dot(gin, f1w_ref[...]) + f1b_ref[...], 0.0)
            g = jax.nn.sigmoid(dot(gpre, f2w_ref[...]) + f2b_ref[...])
            return ln(g * zg + (1.0 - g) * za, flg_ref[...], flb_ref[...])

        z1 = fuse(z1g, za1)
        z2 = fuse(z2g, za2)
        h12 = jnp.maximum(dot(jnp.concatenate([z1, z2], axis=1), g121w_ref[...])
                          + g121b_ref[...], 0.0)
        g12 = jax.nn.sigmoid(dot(h12, g122w_ref[...]) + g122b_ref[...])
        h21 = jnp.maximum(dot(jnp.concatenate([z2, z1], axis=1), g211w_ref[...])
                          + g211b_ref[...], 0.0)
        g21 = jax.nn.sigmoid(dot(h21, g212w_ref[...]) + g212b_ref[...])
        z1n = ln(z1 + g12 * z2, c1g_ref[...], c1b_ref[...])
        z2n = ln(z2 + g21 * z1, c2g_ref[...], c2b_ref[...])
        feat = jnp.concatenate(
            [z1n, z2n, jnp.abs(z1n - z2n), z1n * z2n], axis=1)
        hsh = jnp.maximum(dot(feat, s1w_ref[...]) + s1b_ref[...], 0.0)
        hsh = jnp.maximum(dot(hsh, s2w_ref[...]) + s2b_ref[...], 0.0)
        logits_ref[...] = dot(hsh, lw_ref[...]) + lb_ref[...]
        cd = jnp.maximum(dot(hsh, cd1w_ref[...]) + cd1b_ref[...], 0.0)
        cd = dot(cd, cd2w_ref[...]) + cd2b_ref[...]
        card_ref[...] = (jnp.maximum(cd, 0.0)
                         + jnp.log(1.0 + jnp.exp(-jnp.abs(cd))))

    args = (
        z_mean, z_max, aux1, aux2,
        pk["readout"]["w"].T, pk["readout"]["b"].reshape(1, -1),
        pk["aux1"]["w"].T, pk["aux1"]["b"].reshape(1, -1),
        pk["aux2"]["w"].T, pk["aux2"]["b"].reshape(1, -1),
        pk["fus1"]["w"].T, pk["fus1"]["b"].reshape(1, -1),
        pk["fus2"]["w"].T, pk["fus2"]["b"].reshape(1, -1),
        pk["fus_ln_g"].reshape(1, -1), pk["fus_ln_b"].reshape(1, -1),
        pk["g12_1"]["w"].T, pk["g12_1"]["b"].reshape(1, -1),
        pk["g12_2"]["w"].T, pk["g12_2"]["b"].reshape(1, -1),
        pk["g21_1"]["w"].T, pk["g21_1"]["b"].reshape(1, -1),
        pk["g21_2"]["w"].T, pk["g21_2"]["b"].reshape(1, -1),
        pk["cg_ln1_g"].reshape(1, -1), pk["cg_ln1_b"].reshape(1, -1),
        pk["cg_ln2_g"].reshape(1, -1), pk["cg_ln2_b"].reshape(1, -1),
        pk["sh1"]["w"].T, pk["sh1"]["b"].reshape(1, -1),
        pk["sh2"]["w"].T, pk["sh2"]["b"].reshape(1, -1),
        pk["label"]["w"].T, pk["label"]["b"].reshape(1, -1),
        pk["card1"]["w"].T, pk["card1"]["b"].reshape(1, -1),
        pk["card2"]["w"].T, pk["card2"]["b"].reshape(1, -1),
    )
    return pl.pallas_call(
        body,
        out_shape=[
            jax.ShapeDtypeStruct((NG, 86), jnp.float32),
            jax.ShapeDtypeStruct((NG, 1), jnp.float32),
        ],
    )(*args)


def kernel(g1_x, g1_edge_index, g1_edge_attr, g1_batch,
           g2_x, g2_edge_index, g2_edge_attr, g2_batch,
           aux1, aux2, n_graphs, params):
    f32 = jnp.float32
    x2 = jnp.concatenate([g1_x, g2_x], axis=0)
    pad_i = jnp.zeros((PADE,), jnp.int32)
    pad_d = jnp.full((PADE,), N, jnp.int32)
    pad_e = jnp.zeros((PADE, EIN), f32)
    src2 = jnp.concatenate(
        [g1_edge_index[0], pad_i, g2_edge_index[0] + N, pad_i])
    gidx = src2 >> 1
    gpar = (src2 & 1).reshape(E2, 1)
    dst2 = jnp.concatenate(
        [g1_edge_index[1], pad_d, g2_edge_index[1], pad_d]).reshape(
            2, 16, EG // 16 // CH, CH)
    ea2 = jnp.concatenate([g1_edge_attr, pad_e, g2_edge_attr, pad_e], axis=0)
    batch3d = jnp.concatenate([g1_batch, g2_batch + NG]).reshape(20, 1, 1000)

    p = params
    h, h_bf = _tc_linear_relu(x2, p["lin_in"]["w"].T, p["lin_in"]["b"])
    for lp in p["layers"]:
        w1 = lp["msg1"]["w"]
        hs = _sc_gather(h_bf.reshape(N2 // 2, H), gidx)
        m = _tc_msg_mlp(hs, gpar, ea2, w1[:, :H].T, w1[:, H:].T,
                        lp["msg1"]["b"], lp["msg2"]["w"].T, lp["msg2"]["b"])
        aggp = _sc_scatter(m, dst2)
        agg = aggp[:, :N, :].reshape(N2, H)
        inv = lp["bn_gamma"] * lax.rsqrt(lp["bn_var"] + 1e-5)
        shift = lp["bn_beta"] - lp["bn_mean"] * inv
        h, h_bf = _tc_gru(agg, h, lp["gru_wih"].T, lp["gru_whh"].T,
                          lp["gru_bih"], lp["gru_bhh"], inv, shift)

    z_mean, z_max = _tc_readout(h, batch3d)
    logits, card = _tc_head(z_mean, z_max, aux1, aux2, p)
    zz = (jnp.asarray(n_graphs) * 0).astype(f32)
    return (logits + zz, card + zz)


# speedup vs baseline: 1.1489x; 1.1489x over previous
import functools

import jax
import jax.numpy as jnp
from jax import lax
from jax.experimental import pallas as pl
from jax.experimental.pallas import tpu as pltpu
from jax.experimental.pallas import tpu_sc as plsc

N = 10000
H = 128
EIN = 16
E = 320000
PADE = 7680
EG = E + PADE
E2 = 2 * EG
N2 = 2 * N
NPAD = 10240
NG = 256
NW = 32
CH = 128


NB = 5
LAG = NB - 2


def _sc_gather(h2, idx):
    mesh = plsc.VectorSubcoreMesh(core_axis_name="c", subcore_axis_name="s")
    epw = E2 // NW
    nch = epw // CH

    dt = h2.dtype
    kw = h2.shape[1]

    def body(h_hbm, idx_hbm, out_hbm, idx_all, rows, gsem, osem):
        wid = lax.axis_index("s") * 2 + lax.axis_index("c")
        base = wid * epw
        pltpu.sync_copy(idx_hbm.at[pl.ds(base, epw)], idx_all)

        def gstart(p):
            b = lax.rem(p, NB)
            pltpu.async_copy(h_hbm.at[idx_all.at[pl.ds(p * CH, CH)]],
                             rows.at[b], gsem.at[b])

        def gwait(p):
            b = lax.rem(p, NB)
            pltpu.make_async_copy(h_hbm.at[pl.ds(0, CH)], rows.at[b],
                                  gsem.at[b]).wait()

        def ostart(p):
            b = lax.rem(p, NB)
            pltpu.async_copy(rows.at[b],
                             out_hbm.at[pl.ds(base + p * CH, CH)],
                             osem.at[b])

        def owait(p):
            b = lax.rem(p, NB)
            pltpu.make_async_copy(rows.at[b],
                                  out_hbm.at[pl.ds(base + p * CH, CH)],
                                  osem.at[b]).wait()

        def step(p, carry):
            @pl.when(p < nch)
            def _():
                @pl.when(p >= NB)
                def _():
                    owait(p - NB)
                gstart(p)

            @pl.when(p >= LAG)
            def _():
                gwait(p - LAG)
                ostart(p - LAG)
            return carry

        lax.fori_loop(0, nch + LAG, step, 0)
        for q in range(nch - NB, nch):
            owait(q)

    f = pl.kernel(
        body,
        out_type=jax.ShapeDtypeStruct((E2, kw), dt),
        mesh=mesh,
        scratch_types=[
            pltpu.VMEM((epw,), jnp.int32),
            pltpu.VMEM((NB, CH, kw), dt),
            pltpu.SemaphoreType.DMA((NB,)),
            pltpu.SemaphoreType.DMA((NB,)),
        ],
    )
    return f(h2, idx)


def _sc_scatter(m, dst):
    mesh = plsc.VectorSubcoreMesh(core_axis_name="c", subcore_axis_name="s")
    ept = EG // 16
    nch = ept // CH
    rpt = NPAD // 16

    nb = 2

    def body(m_hbm, dst_hbm, out_hbm, idxb, mbuf, zbuf, acc,
             isem, lsem, ssem):
        c = lax.axis_index("c")
        s = lax.axis_index("s")

        def zb(i, carry):
            zbuf[i // 8, pl.ds((i % 8) * 16, 16)] = jnp.zeros((16,), jnp.float32)
            return carry

        lax.fori_loop(0, 32 * 8, zb, 0)

        def za(i, carry):
            pltpu.sync_copy(zbuf, acc.at[pl.ds(s * rpt + i * 32, 32)])
            return carry

        lax.fori_loop(0, rpt // 32, za, 0)
        plsc.subcore_barrier()

        base = c * EG + s * ept

        def istart(p):
            b = lax.rem(p, nb)
            pltpu.async_copy(dst_hbm.at[c, s, p], idxb.at[b], isem.at[b])

        def iwait(p):
            b = lax.rem(p, nb)
            pltpu.make_async_copy(dst_hbm.at[c, s, p], idxb.at[b],
                                  isem.at[b]).wait()

        def lstart(p):
            b = lax.rem(p, nb)
            pltpu.async_copy(m_hbm.at[pl.ds(base + p * CH, CH)],
                             mbuf.at[b], lsem.at[b])

        def lwait(p):
            b = lax.rem(p, nb)
            pltpu.make_async_copy(m_hbm.at[pl.ds(base + p * CH, CH)],
                                  mbuf.at[b], lsem.at[b]).wait()

        def sstart(p):
            b = lax.rem(p, nb)
            pltpu.async_copy(mbuf.at[b], acc.at[idxb.at[b]],
                             ssem.at[b], add=True)

        def swait(p):
            b = lax.rem(p, nb)
            pltpu.make_async_copy(mbuf.at[b], acc.at[pl.ds(0, CH)],
                                  ssem.at[b]).wait()

        def step(p, carry):
            @pl.when(p < nch)
            def _():
                @pl.when(p >= nb)
                def _():
                    swait(p - nb)
                istart(p)
                lstart(p)

            @pl.when(p >= 1)
            def _():
                iwait(p - 1)
                lwait(p - 1)
                sstart(p - 1)
            return carry

        lax.fori_loop(0, nch + 1, step, 0)
        swait(nch - 2)
        swait(nch - 1)
        plsc.subcore_barrier()

        pltpu.sync_copy(acc.at[pl.ds(s * rpt, rpt)],
                        out_hbm.at[c, pl.ds(s * rpt, rpt)])

    f = pl.kernel(
        body,
        out_type=jax.ShapeDtypeStruct((2, NPAD, H), jnp.float32),
        mesh=mesh,
        scratch_types=[
            pltpu.VMEM((nb, CH), jnp.int32),
            pltpu.VMEM((nb, CH, H), jnp.float32),
            pltpu.VMEM((32, H), jnp.float32),
            pltpu.VMEM_SHARED((NPAD, H), jnp.float32),
            pltpu.SemaphoreType.DMA((nb,)),
            pltpu.SemaphoreType.DMA((nb,)),
            pltpu.SemaphoreType.DMA((nb,)),
        ],
    )
    return f(m, dst)


def _tc_linear_relu(x, wt, b):
    bm = 2000
    grid = x.shape[0] // bm

    def body(x_ref, w_ref, b_ref, o_ref):
        o_ref[...] = jnp.maximum(
            jnp.dot(x_ref[...], w_ref[...], preferred_element_type=jnp.float32)
            + b_ref[...], 0.0)

    return pl.pallas_call(
        body,
        grid=(grid,),
        in_specs=[
            pl.BlockSpec((bm, x.shape[1]), lambda i: (i, 0)),
            pl.BlockSpec(wt.shape, lambda i: (0, 0)),
            pl.BlockSpec((1, wt.shape[1]), lambda i: (0, 0)),
        ],
        out_specs=pl.BlockSpec((bm, wt.shape[1]), lambda i: (i, 0)),
        out_shape=jax.ShapeDtypeStruct((x.shape[0], wt.shape[1]), jnp.float32),
    )(x, wt, b.reshape(1, -1))


def _tc_msg_mlp(hs, ea, w1h, w1e, b1, w2, b2):
    bm = 4096
    grid = E2 // bm

    def body(hs_ref, ea_ref, w1h_ref, w1e_ref, b1_ref, w2_ref,
             b2_ref, o_ref):
        m1 = jnp.dot(hs_ref[...], w1h_ref[...],
                     preferred_element_type=jnp.float32)
        m1 = m1 + jnp.dot(ea_ref[...], w1e_ref[...],
                          preferred_element_type=jnp.float32)
        m1 = jnp.maximum(m1 + b1_ref[...], 0.0)
        o_ref[...] = (jnp.dot(m1, w2_ref[...], preferred_element_type=jnp.float32)
                      + b2_ref[...])

    return pl.pallas_call(
        body,
        grid=(grid,),
        in_specs=[
            pl.BlockSpec((bm, H), lambda i: (i, 0)),
            pl.BlockSpec((bm, EIN), lambda i: (i, 0)),
            pl.BlockSpec((H, H), lambda i: (0, 0)),
            pl.BlockSpec((EIN, H), lambda i: (0, 0)),
            pl.BlockSpec((1, H), lambda i: (0, 0)),
            pl.BlockSpec((H, H), lambda i: (0, 0)),
            pl.BlockSpec((1, H), lambda i: (0, 0)),
        ],
        out_specs=pl.BlockSpec((bm, H), lambda i: (i, 0)),
        out_shape=jax.ShapeDtypeStruct((E2, H), jnp.float32),
    )(hs, ea, w1h, w1e, b1.reshape(1, H), w2, b2.reshape(1, H))


def _tc_gru(agg, h, wih_t, whh_t, bih, bhh, bn_scale, bn_shift):
    bm = 2000
    grid = N2 // bm

    def body(agg_ref, h_ref, wih_ref, whh_ref, bih_ref, bhh_ref,
             bs_ref, bb_ref, o_ref):
        hv = h_ref[...]
        gi = (jnp.dot(agg_ref[...], wih_ref[...],
                      preferred_element_type=jnp.float32) + bih_ref[...])
        gh = (jnp.dot(hv, whh_ref[...],
                      preferred_element_type=jnp.float32) + bhh_ref[...])
        r = jax.nn.sigmoid(gi[:, :H] + gh[:, :H])
        z = jax.nn.sigmoid(gi[:, H:2 * H] + gh[:, H:2 * H])
        nn = jnp.tanh(gi[:, 2 * H:] + r * gh[:, 2 * H:])
        hn = (1.0 - z) * nn + z * hv
        o_ref[...] = hv + hn * bs_ref[...] + bb_ref[...]

    return pl.pallas_call(
        body,
        grid=(grid,),
        in_specs=[
            pl.BlockSpec((bm, H), lambda i: (i, 0)),
            pl.BlockSpec((bm, H), lambda i: (i, 0)),
            pl.BlockSpec((H, 3 * H), lambda i: (0, 0)),
            pl.BlockSpec((H, 3 * H), lambda i: (0, 0)),
            pl.BlockSpec((1, 3 * H), lambda i: (0, 0)),
            pl.BlockSpec((1, 3 * H), lambda i: (0, 0)),
            pl.BlockSpec((1, H), lambda i: (0, 0)),
            pl.BlockSpec((1, H), lambda i: (0, 0)),
        ],
        out_specs=pl.BlockSpec((bm, H), lambda i: (i, 0)),
        out_shape=jax.ShapeDtypeStruct((N2, H), jnp.float32),
    )(agg, h, wih_t, whh_t, bih.reshape(1, -1), bhh.reshape(1, -1),
      bn_scale.reshape(1, H), bn_shift.reshape(1, H))


def _tc_readout(h, batch3d):
    bn = 1000
    grid = N2 // bn
    nseg = 2 * NG

    def body(h_ref, b_ref, bc_ref, zmean_ref, zmax_ref, accs, accm, accc):
        i = pl.program_id(0)

        @pl.when(i == 0)
        def _init():
            accs[...] = jnp.zeros_like(accs)
            accm[...] = jnp.full_like(accm, -1e30)
            accc[...] = jnp.zeros_like(accc)

        bv = b_ref[...].reshape(1, bn)
        hv = h_ref[...]
        seg = lax.broadcasted_iota(jnp.int32, (nseg, bn), 0)
        p = (seg == bv).astype(jnp.float32)
        accs[...] += jnp.dot(p, hv, preferred_element_type=jnp.float32)
        accc[...] += jnp.sum(p, axis=1, keepdims=True)

        g0 = jnp.min(bv)
        g1 = jnp.max(bv)

        bc = bc_ref[...]

        def gbody(g, carry):
            mask = bc == g
            hm = jnp.max(jnp.where(mask, hv, -1e30), axis=0, keepdims=True)
            accm[pl.ds(g, 1), :] = jnp.maximum(accm[pl.ds(g, 1), :], hm)
            return carry

        lax.fori_loop(g0, g1 + 1, gbody, 0)

        @pl.when(i == grid - 1)
        def _fin():
            cnt = accc[...]
            zmean_ref[...] = accs[...] / jnp.maximum(cnt, 1.0)
            zmax_ref[...] = jnp.where(cnt > 0.0, accm[...], 0.0)

    return pl.pallas_call(
        body,
        grid=(grid,),
        in_specs=[
            pl.BlockSpec((bn, H), lambda i: (i, 0)),
            pl.BlockSpec((1, 1, bn), lambda i: (i, 0, 0)),
            pl.BlockSpec((bn, 1), lambda i: (i, 0)),
        ],
        out_specs=[
            pl.BlockSpec((nseg, H), lambda i: (0, 0)),
            pl.BlockSpec((nseg, H), lambda i: (0, 0)),
        ],
        out_shape=[
            jax.ShapeDtypeStruct((nseg, H), jnp.float32),
            jax.ShapeDtypeStruct((nseg, H), jnp.float32),
        ],
        scratch_shapes=[
            pltpu.VMEM((nseg, H), jnp.float32),
            pltpu.VMEM((nseg, H), jnp.float32),
            pltpu.VMEM((nseg, 1), jnp.float32),
        ],
    )(h, batch3d, batch3d.reshape(N2, 1))


def _tc_head(z_mean, z_max, aux1, aux2, pk):
    def ln(x, g, b):
        mu = jnp.mean(x, axis=-1, keepdims=True)
        v = jnp.mean((x - mu) ** 2, axis=-1, keepdims=True)
        return (x - mu) * lax.rsqrt(v + 1e-5) * g + b

    def dot(a, b):
        return jnp.dot(a, b, preferred_element_type=jnp.float32)

    def body(zm_ref, zx_ref, a1_ref, a2_ref,
             wr_ref, br_ref, aw1_ref, ab1_ref, aw2_ref, ab2_ref,
             f1w_ref, f1b_ref, f2w_ref, f2b_ref, flg_ref, flb_ref,
             g121w_ref, g121b_ref, g122w_ref, g122b_ref,
             g211w_ref, g211b_ref, g212w_ref, g212b_ref,
             c1g_ref, c1b_ref, c2g_ref, c2b_ref,
             s1w_ref, s1b_ref, s2w_ref, s2b_ref,
             lw_ref, lb_ref, cd1w_ref, cd1b_ref, cd2w_ref, cd2b_ref,
             logits_ref, card_ref):
        zcat = jnp.concatenate([zm_ref[...], zx_ref[...]], axis=1)
        z = jnp.maximum(dot(zcat, wr_ref[...]) + br_ref[...], 0.0)
        z1g = z[:NG]
        z2g = z[NG:]
        za1 = jnp.maximum(dot(a1_ref[...], aw1_ref[...]) + ab1_ref[...], 0.0)
        za1 = jnp.maximum(dot(za1, aw2_ref[...]) + ab2_ref[...], 0.0)
        za2 = jnp.maximum(dot(a2_ref[...], aw1_ref[...]) + ab1_ref[...], 0.0)
        za2 = jnp.maximum(dot(za2, aw2_ref[...]) + ab2_ref[...], 0.0)

        def fuse(zg, za):
            gin = jnp.concatenate([zg, za], axis=1)
            gpre = jnp.maximum(dot(gin, f1w_ref[...]) + f1b_ref[...], 0.0)
            g = jax.nn.sigmoid(dot(gpre, f2w_ref[...]) + f2b_ref[...])
            return ln(g * zg + (1.0 - g) * za, flg_ref[...], flb_ref[...])

        z1 = fuse(z1g, za1)
        z2 = fuse(z2g, za2)
        h12 = jnp.maximum(dot(jnp.concatenate([z1, z2], axis=1), g121w_ref[...])
                          + g121b_ref[...], 0.0)
        g12 = jax.nn.sigmoid(dot(h12, g122w_ref[...]) + g122b_ref[...])
        h21 = jnp.maximum(dot(jnp.concatenate([z2, z1], axis=1), g211w_ref[...])
                          + g211b_ref[...], 0.0)
        g21 = jax.nn.sigmoid(dot(h21, g212w_ref[...]) + g212b_ref[...])
        z1n = ln(z1 + g12 * z2, c1g_ref[...], c1b_ref[...])
        z2n = ln(z2 + g21 * z1, c2g_ref[...], c2b_ref[...])
        feat = jnp.concatenate(
            [z1n, z2n, jnp.abs(z1n - z2n), z1n * z2n], axis=1)
        hsh = jnp.maximum(dot(feat, s1w_ref[...]) + s1b_ref[...], 0.0)
        hsh = jnp.maximum(dot(hsh, s2w_ref[...]) + s2b_ref[...], 0.0)
        logits_ref[...] = dot(hsh, lw_ref[...]) + lb_ref[...]
        cd = jnp.maximum(dot(hsh, cd1w_ref[...]) + cd1b_ref[...], 0.0)
        cd = dot(cd, cd2w_ref[...]) + cd2b_ref[...]
        card_ref[...] = (jnp.maximum(cd, 0.0)
                         + jnp.log(1.0 + jnp.exp(-jnp.abs(cd))))

    args = (
        z_mean, z_max, aux1, aux2,
        pk["readout"]["w"].T, pk["readout"]["b"].reshape(1, -1),
        pk["aux1"]["w"].T, pk["aux1"]["b"].reshape(1, -1),
        pk["aux2"]["w"].T, pk["aux2"]["b"].reshape(1, -1),
        pk["fus1"]["w"].T, pk["fus1"]["b"].reshape(1, -1),
        pk["fus2"]["w"].T, pk["fus2"]["b"].reshape(1, -1),
        pk["fus_ln_g"].reshape(1, -1), pk["fus_ln_b"].reshape(1, -1),
        pk["g12_1"]["w"].T, pk["g12_1"]["b"].reshape(1, -1),
        pk["g12_2"]["w"].T, pk["g12_2"]["b"].reshape(1, -1),
        pk["g21_1"]["w"].T, pk["g21_1"]["b"].reshape(1, -1),
        pk["g21_2"]["w"].T, pk["g21_2"]["b"].reshape(1, -1),
        pk["cg_ln1_g"].reshape(1, -1), pk["cg_ln1_b"].reshape(1, -1),
        pk["cg_ln2_g"].reshape(1, -1), pk["cg_ln2_b"].reshape(1, -1),
        pk["sh1"]["w"].T, pk["sh1"]["b"].reshape(1, -1),
        pk["sh2"]["w"].T, pk["sh2"]["b"].reshape(1, -1),
        pk["label"]["w"].T, pk["label"]["b"].reshape(1, -1),
        pk["card1"]["w"].T, pk["card1"]["b"].reshape(1, -1),
        pk["card2"]["w"].T, pk["card2"]["b"].reshape(1, -1),
    )
    return pl.pallas_call(
        body,
        out_shape=[
            jax.ShapeDtypeStruct((NG, 86), jnp.float32),
            jax.ShapeDtypeStruct((NG, 1), jnp.float32),
        ],
    )(*args)


def kernel(g1_x, g1_edge_index, g1_edge_attr, g1_batch,
           g2_x, g2_edge_index, g2_edge_attr, g2_batch,
           aux1, aux2, n_graphs, params):
    f32 = jnp.float32
    x2 = jnp.concatenate([g1_x, g2_x], axis=0)
    pad_i = jnp.zeros((PADE,), jnp.int32)
    pad_d = jnp.full((PADE,), N, jnp.int32)
    pad_e = jnp.zeros((PADE, EIN), f32)
    src2 = jnp.concatenate(
        [g1_edge_index[0], pad_i, g2_edge_index[0] + N, pad_i])
    dst2 = jnp.concatenate(
        [g1_edge_index[1], pad_d, g2_edge_index[1], pad_d]).reshape(
            2, 16, EG // 16 // CH, CH)
    ea2 = jnp.concatenate([g1_edge_attr, pad_e, g2_edge_attr, pad_e], axis=0)
    batch3d = jnp.concatenate([g1_batch, g2_batch + NG]).reshape(20, 1, 1000)

    p = params
    h = _tc_linear_relu(x2, p["lin_in"]["w"].T, p["lin_in"]["b"])
    for lp in p["layers"]:
        w1 = lp["msg1"]["w"]
        hs = _sc_gather(h, src2)
        m = _tc_msg_mlp(hs, ea2, w1[:, :H].T, w1[:, H:].T,
                        lp["msg1"]["b"], lp["msg2"]["w"].T, lp["msg2"]["b"])
        aggp = _sc_scatter(m, dst2)
        agg = aggp[:, :N, :].reshape(N2, H)
        inv = lp["bn_gamma"] * lax.rsqrt(lp["bn_var"] + 1e-5)
        shift = lp["bn_beta"] - lp["bn_mean"] * inv
        h = _tc_gru(agg, h, lp["gru_wih"].T, lp["gru_whh"].T,
                    lp["gru_bih"], lp["gru_bhh"], inv, shift)

    z_mean, z_max = _tc_readout(h, batch3d)
    logits, card = _tc_head(z_mean, z_max, aux1, aux2, p)
    zz = (jnp.asarray(n_graphs) * 0).astype(f32)
    return (logits + zz, card + zz)
